# baseline (speedup 1.0000x reference)
import functools

import jax
import jax.numpy as jnp
from jax import lax
from jax.experimental import pallas as pl
from jax.experimental.pallas import tpu as pltpu
from jax.experimental.pallas import tpu_sc as plsc

N_EMBEDDINGS = 1000000
D_EMBEDDING = 16

_NC = 2
_NS = 16
_NW = _NC * _NS

_CHUNK = 128


def _quantize_body(x_ref, idx_ref):
    x = x_ref[...]
    m = jnp.max(x, axis=1, keepdims=True)
    xn = x / m * float(N_EMBEDDINGS - 1)
    xn = jnp.where(xn < 0, 0.0, xn)
    idx_ref[...] = xn.astype(jnp.int32)


def _quantize(x):
    return pl.pallas_call(
        _quantize_body,
        out_shape=jax.ShapeDtypeStruct(x.shape, jnp.int32),
    )(x)


def _make_gather(n_rows):
    assert n_rows % (_NW * _CHUNK) == 0
    bpw = n_rows // _NW
    nchunk = bpw // _CHUNK
    mesh = plsc.VectorSubcoreMesh(core_axis_name="c", subcore_axis_name="s")

    @functools.partial(
        pl.kernel,
        mesh=mesh,
        out_type=jax.ShapeDtypeStruct((n_rows, D_EMBEDDING), jnp.float32),
        compiler_params=pltpu.CompilerParams(use_tc_tiling_on_sc=False),
        scratch_types=[
            pltpu.VMEM((_CHUNK,), jnp.int32),
            pltpu.VMEM((_CHUNK, D_EMBEDDING), jnp.float32),
            pltpu.SemaphoreType.DMA,
        ],
    )
    def gather(table_hbm, idx_hbm, out_hbm, idx_v, rows_v, sem):
        wid = lax.axis_index("s") * _NC + lax.axis_index("c")
        base = wid * bpw

        def body(i, carry):
            off = base + i * _CHUNK
            pltpu.sync_copy(idx_hbm.at[pl.ds(off, _CHUNK)], idx_v)
            pltpu.async_copy(table_hbm.at[idx_v], rows_v, sem).wait()
            pltpu.sync_copy(rows_v, out_hbm.at[pl.ds(off, _CHUNK)])
            return carry

        lax.fori_loop(0, nchunk, body, 0)

    return gather


def kernel(x, table):
    b, s = x.shape
    idx = _quantize(x).reshape(b * s)
    flat = _make_gather(b * s)(table, idx)
    return flat.reshape(b, s, D_EMBEDDING)


# speedup vs baseline: 1.2140x; 1.2140x over previous
import functools

import jax
import jax.numpy as jnp
from jax import lax
from jax.experimental import pallas as pl
from jax.experimental.pallas import tpu as pltpu
from jax.experimental.pallas import tpu_sc as plsc

N_EMBEDDINGS = 1000000
D_EMBEDDING = 16

_NC = 2
_NS = 16
_NW = _NC * _NS

_CHUNK = 128
_K = 10
_GROUP = _K * _CHUNK


def _quantize_body(x_ref, idx_ref):
    x = x_ref[...]
    m = jnp.max(x, axis=1, keepdims=True)
    xn = x / m * float(N_EMBEDDINGS - 1)
    xn = jnp.where(xn < 0, 0.0, xn)
    idx_ref[...] = xn.astype(jnp.int32)


def _quantize(x):
    return pl.pallas_call(
        _quantize_body,
        out_shape=jax.ShapeDtypeStruct(x.shape, jnp.int32),
    )(x)


def _make_gather(n_rows):
    assert n_rows % (_NW * _GROUP) == 0
    bpw = n_rows // _NW
    nchunk = bpw // _CHUNK
    ngroup = bpw // _GROUP
    assert ngroup % 2 == 0
    mesh = plsc.VectorSubcoreMesh(core_axis_name="c", subcore_axis_name="s")

    @functools.partial(
        pl.kernel,
        mesh=mesh,
        out_type=jax.ShapeDtypeStruct((n_rows, D_EMBEDDING), jnp.float32),
        compiler_params=pltpu.CompilerParams(use_tc_tiling_on_sc=False),
        scratch_types=[
            pltpu.VMEM((nchunk, _CHUNK), jnp.int32),
            pltpu.VMEM((_GROUP, D_EMBEDDING), jnp.float32),
            pltpu.VMEM((_GROUP, D_EMBEDDING), jnp.float32),
            pltpu.SemaphoreType.DMA,
            pltpu.SemaphoreType.DMA,
            pltpu.SemaphoreType.DMA,
            pltpu.SemaphoreType.DMA,
        ],
    )
    def gather(table_hbm, idx_hbm, out_hbm, idx_v, buf0, buf1,
               gsem0, gsem1, wsem0, wsem1):
        wid = lax.axis_index("s") * _NC + lax.axis_index("c")
        base = wid * bpw

        pltpu.sync_copy(idx_hbm.at[pl.ds(wid * nchunk, nchunk)], idx_v)

        def process(g, buf, gsem, wsem, first):
            if not first:
                pltpu.make_async_copy(
                    buf, out_hbm.at[pl.ds(base, _GROUP)], wsem).wait()
            descs = []
            for j in range(_K):
                descs.append(pltpu.async_copy(
                    table_hbm.at[idx_v.at[g * _K + j]],
                    buf.at[pl.ds(j * _CHUNK, _CHUNK)],
                    gsem))
            for d in descs:
                d.wait()
            pltpu.async_copy(
                buf, out_hbm.at[pl.ds(base + g * _GROUP, _GROUP)], wsem)

        process(0, buf0, gsem0, wsem0, first=True)
        process(1, buf1, gsem1, wsem1, first=True)

        def body(i, carry):
            process(2 * i, buf0, gsem0, wsem0, first=False)
            process(2 * i + 1, buf1, gsem1, wsem1, first=False)
            return carry

        lax.fori_loop(1, ngroup // 2, body, 0)

        pltpu.make_async_copy(buf0, out_hbm.at[pl.ds(base, _GROUP)], wsem0).wait()
        pltpu.make_async_copy(buf1, out_hbm.at[pl.ds(base, _GROUP)], wsem1).wait()

    return gather


def kernel(x, table):
    b, s = x.shape
    n = b * s
    idx = _quantize(x).reshape(n // _CHUNK, _CHUNK)
    flat = _make_gather(n)(table, idx)
    return flat.reshape(b, s, D_EMBEDDING)


# speedup vs baseline: 1.5199x; 1.2520x over previous
import functools

import jax
import jax.numpy as jnp
from jax import lax
from jax.experimental import pallas as pl
from jax.experimental.pallas import tpu as pltpu
from jax.experimental.pallas import tpu_sc as plsc

N_EMBEDDINGS = 1000000
D_EMBEDDING = 16

_NC = 2
_NS = 16
_NW = _NC * _NS

_B = 4096
_S = 200
_LANE = 128
_GS = 4
_NG = _S // _GS


def _quantize_body(x_ref, idx_ref):
    x = x_ref[...]
    m = jnp.max(x, axis=1, keepdims=True)
    xn = x / m * float(N_EMBEDDINGS - 1)
    xn = jnp.where(xn < 0, 0.0, xn)
    idx_ref[...] = xn.astype(jnp.int32).T


def _quantize_t(x):
    return pl.pallas_call(
        _quantize_body,
        out_shape=jax.ShapeDtypeStruct((_S, _B), jnp.int32),
    )(x)


def _make_gather():
    mesh = plsc.VectorSubcoreMesh(core_axis_name="c", subcore_axis_name="s")
    n_out = _S * D_EMBEDDING * _B

    @functools.partial(
        pl.kernel,
        mesh=mesh,
        out_type=jax.ShapeDtypeStruct((n_out,), jnp.float32),
        compiler_params=pltpu.CompilerParams(
            use_tc_tiling_on_sc=False, needs_layout_passes=False),
        scratch_types=[
            pltpu.VMEM((_S, _LANE), jnp.int32),
            pltpu.VMEM((_GS * _LANE, D_EMBEDDING), jnp.float32),
            pltpu.VMEM((_GS * _LANE, D_EMBEDDING), jnp.float32),
            pltpu.VMEM((_GS * 2048,), jnp.float32),
            pltpu.VMEM((_GS * 2048,), jnp.float32),
            pltpu.SemaphoreType.DMA,
            pltpu.SemaphoreType.DMA,
        ],
    )
    def gather(table_hbm, idxt_hbm, out_hbm, idxv, r_a, r_b, stg_a, stg_b,
               gsem, wsem):
        wid = lax.axis_index("s") * _NC + lax.axis_index("c")
        iota = lax.iota(jnp.int32, 16)

        pltpu.sync_copy(idxt_hbm.at[:, pl.ds(wid * _LANE, _LANE)], idxv)

        def fire_g(g, rbuf):
            for j in range(_GS):
                pltpu.async_copy(
                    table_hbm.at[idxv.at[g * _GS + j]],
                    rbuf.at[pl.ds(j * _LANE, _LANE)], gsem)

        def drain_g(rbuf):
            for j in range(_GS):
                pltpu.make_async_copy(
                    table_hbm.at[idxv.at[0]],
                    rbuf.at[pl.ds(j * _LANE, _LANE)], gsem).wait()

        def transpose(rbuf, stg):
            for j in range(_GS):
                for d in range(D_EMBEDDING):
                    col = jnp.full((16,), d, jnp.int32)
                    for c in range(8):
                        row = iota + (j * _LANE + c * 16)
                        v = plsc.load_gather(rbuf, [row, col])
                        stg[pl.ds(j * 2048 + d * 128 + c * 16, 16)] = v

        def fire_w(g, stg):
            for j in range(_GS):
                s = g * _GS + j
                for dt in range(2):
                    pltpu.async_copy(
                        stg.at[pl.ds(j * 2048 + dt * 1024, 1024)],
                        out_hbm.at[pl.ds(((2 * s + dt) * 32 + wid) * 1024, 1024)],
                        wsem)

        def drain_w(stg):
            for j in range(_GS):
                for dt in range(2):
                    pltpu.make_async_copy(
                        stg.at[pl.ds(j * 2048 + dt * 1024, 1024)],
                        out_hbm.at[pl.ds(0, 1024)], wsem).wait()

        def process(g, rbuf, stg, first):
            if not first:
                drain_w(stg)
            fire_g(g, rbuf)
            drain_g(rbuf)
            transpose(rbuf, stg)
            fire_w(g, stg)

        process(0, r_a, stg_a, first=True)
        process(1, r_b, stg_b, first=True)

        def body(i, carry):
            process(2 * i, r_a, stg_a, first=False)
            process(2 * i + 1, r_b, stg_b, first=False)
            return carry

        lax.fori_loop(1, _NG // 2, body, 0)
        drain_w(stg_a)
        drain_w(stg_b)

    return gather


def kernel(x, table):
    idx_t = _quantize_t(x)
    flat = _make_gather()(table, idx_t)
    return (flat.reshape(_S, 2, 32, 8, _LANE)
            .transpose(2, 4, 0, 1, 3)
            .reshape(_B, _S, D_EMBEDDING))


# speedup vs baseline: 2.0053x; 1.3194x over previous
import functools

import jax
import jax.numpy as jnp
from jax import lax
from jax.experimental import pallas as pl
from jax.experimental.pallas import tpu as pltpu
from jax.experimental.pallas import tpu_sc as plsc

N_EMBEDDINGS = 1000000
D_EMBEDDING = 16

_NC = 2
_NS = 16
_NW = _NC * _NS

_B = 4096
_S = 200
_LANE = 128
_GS = 4
_NG = _S // _GS


def _quantize_body(x_ref, idx_ref):
    x = x_ref[...]
    m = jnp.max(x, axis=1, keepdims=True)
    xn = x / m * float(N_EMBEDDINGS - 1)
    xn = jnp.where(xn < 0, 0.0, xn)
    idx_ref[...] = xn.astype(jnp.int32).T


def _quantize_t(x):
    return pl.pallas_call(
        _quantize_body,
        out_shape=jax.ShapeDtypeStruct((_S, _B), jnp.int32),
    )(x)


def _make_gather():
    mesh = plsc.VectorSubcoreMesh(core_axis_name="c", subcore_axis_name="s")
    n_out = _S * D_EMBEDDING * _B

    @functools.partial(
        pl.kernel,
        mesh=mesh,
        out_type=jax.ShapeDtypeStruct((n_out,), jnp.float32),
        compiler_params=pltpu.CompilerParams(
            use_tc_tiling_on_sc=False, needs_layout_passes=False),
        scratch_types=[
            pltpu.VMEM((_S, _LANE), jnp.int32),
            pltpu.VMEM((_GS * _LANE, D_EMBEDDING), jnp.float32),
            pltpu.VMEM((_GS * _LANE, D_EMBEDDING), jnp.float32),
            pltpu.VMEM((_GS * 2048,), jnp.float32),
            pltpu.VMEM((_GS * 2048,), jnp.float32),
            pltpu.SemaphoreType.DMA,
            pltpu.SemaphoreType.DMA,
        ],
    )
    def gather(table_hbm, idxt_hbm, out_hbm, idxv, r_a, r_b, stg_a, stg_b,
               gsem, wsem):
        wid = lax.axis_index("s") * _NC + lax.axis_index("c")
        iota = lax.iota(jnp.int32, 16)

        pltpu.sync_copy(idxt_hbm.at[:, pl.ds(wid * _LANE, _LANE)], idxv)

        def fire_g(g, rbuf):
            for j in range(_GS):
                pltpu.async_copy(
                    table_hbm.at[idxv.at[g * _GS + j]],
                    rbuf.at[pl.ds(j * _LANE, _LANE)], gsem)

        def drain_g(rbuf):
            for j in range(_GS):
                pltpu.make_async_copy(
                    table_hbm.at[idxv.at[0]],
                    rbuf.at[pl.ds(j * _LANE, _LANE)], gsem).wait()

        rows = [iota + c * 16 for c in range(8)]

        def transpose(rbuf, stg):
            @plsc.parallel_loop(0, _GS * D_EMBEDDING, 1, unroll=2)
            def _(i):
                j = i >> 4
                d = i & 15
                col = jnp.zeros((16,), jnp.int32) + d
                base = j * 2048 + d * 128
                joff = j << 7
                for c in range(8):
                    v = plsc.load_gather(rbuf, [rows[c] + joff, col])
                    stg[pl.ds(base + c * 16, 16)] = v

        def fire_w(g, stg):
            for j in range(_GS):
                s = g * _GS + j
                for dt in range(2):
                    pltpu.async_copy(
                        stg.at[pl.ds(j * 2048 + dt * 1024, 1024)],
                        out_hbm.at[pl.ds(((2 * s + dt) * 32 + wid) * 1024, 1024)],
                        wsem)

        def drain_w(stg):
            for j in range(_GS):
                for dt in range(2):
                    pltpu.make_async_copy(
                        stg.at[pl.ds(j * 2048 + dt * 1024, 1024)],
                        out_hbm.at[pl.ds(0, 1024)], wsem).wait()

        def process(g, rbuf, stg, first):
            if not first:
                drain_w(stg)
            fire_g(g, rbuf)
            drain_g(rbuf)
            transpose(rbuf, stg)
            fire_w(g, stg)

        process(0, r_a, stg_a, first=True)
        process(1, r_b, stg_b, first=True)

        def body(i, carry):
            process(2 * i, r_a, stg_a, first=False)
            process(2 * i + 1, r_b, stg_b, first=False)
            return carry

        lax.fori_loop(1, _NG // 2, body, 0)
        drain_w(stg_a)
        drain_w(stg_b)

    return gather


def kernel(x, table):
    idx_t = _quantize_t(x)
    flat = _make_gather()(table, idx_t)
    return (flat.reshape(_S, 2, 32, 8, _LANE)
            .transpose(2, 4, 0, 1, 3)
            .reshape(_B, _S, D_EMBEDDING))


# speedup vs baseline: 2.2628x; 1.1284x over previous
import functools

import jax
import jax.numpy as jnp
from jax import lax
from jax.experimental import pallas as pl
from jax.experimental.pallas import tpu as pltpu
from jax.experimental.pallas import tpu_sc as plsc

N_EMBEDDINGS = 1000000
D_EMBEDDING = 16

_NC = 2
_NS = 16
_NW = _NC * _NS

_B = 4096
_S = 200
_LANE = 128
_GS = 4
_NG = _S // _GS


def _quantize_body(x_ref, idx_ref):
    x = x_ref[...]
    m = jnp.max(x, axis=1, keepdims=True)
    xn = x / m * float(N_EMBEDDINGS - 1)
    xn = jnp.where(xn < 0, 0.0, xn)
    idx_ref[...] = xn.astype(jnp.int32).T


def _quantize_t(x):
    return pl.pallas_call(
        _quantize_body,
        out_shape=jax.ShapeDtypeStruct((_S, _B), jnp.int32),
    )(x)


_NTILE = 7813
_NROWS_PAD = _NTILE * 128


def _make_relinearize():
    mesh = plsc.VectorSubcoreMesh(core_axis_name="c", subcore_axis_name="s")
    n_full = 244

    @functools.partial(
        pl.kernel,
        mesh=mesh,
        out_type=jax.ShapeDtypeStruct((_NROWS_PAD * 16,), jnp.float32),
        compiler_params=pltpu.CompilerParams(
            use_tc_tiling_on_sc=True, needs_layout_passes=False),
        scratch_types=[
            pltpu.VMEM((16, 128), jnp.float32),
            pltpu.VMEM((16, 128), jnp.float32),
            pltpu.VMEM((2048,), jnp.float32),
            pltpu.VMEM((2048,), jnp.float32),
            pltpu.SemaphoreType.DMA,
            pltpu.SemaphoreType.DMA,
        ],
    )
    def relin(tt_hbm, lin_hbm, v_a, v_b, o_a, o_b, isem, wsem):
        wid = lax.axis_index("s") * _NC + lax.axis_index("c")
        iota = lax.iota(jnp.int32, 16)

        def fire_in(col, vbuf, width):
            for dt in range(2):
                pltpu.async_copy(
                    tt_hbm.at[pl.ds(dt * 8, 8), pl.ds(col * 128, width)],
                    vbuf.at[pl.ds(dt * 8, 8), pl.ds(0, width)], isem)

        def drain_in(vbuf, width):
            for dt in range(2):
                pltpu.make_async_copy(
                    tt_hbm.at[pl.ds(0, 8), pl.ds(0, width)],
                    vbuf.at[pl.ds(dt * 8, 8), pl.ds(0, width)], isem).wait()

        def transpose(vbuf, obuf, width):
            @plsc.parallel_loop(0, width, 1, unroll=4)
            def _(r0):
                v = plsc.load_gather(vbuf, [iota, jnp.zeros((16,), jnp.int32) + r0])
                obuf[pl.ds(r0 * 16, 16)] = v

        def fire_out(col, obuf, width):
            pltpu.async_copy(obuf.at[pl.ds(0, width * 16)],
                             lin_hbm.at[pl.ds(col * 2048, width * 16)], wsem)

        def drain_out(obuf):
            pltpu.make_async_copy(obuf, lin_hbm.at[pl.ds(0, 2048)], wsem).wait()

        def process(k, vbuf, obuf, first):
            col = k * 32 + wid
            if not first:
                drain_out(obuf)
            fire_in(col, vbuf, 128)
            drain_in(vbuf, 128)
            transpose(vbuf, obuf, 128)
            fire_out(col, obuf, 128)

        process(0, v_a, o_a, first=True)
        process(1, v_b, o_b, first=True)

        def body(i, carry):
            process(2 * i, v_a, o_a, first=False)
            process(2 * i + 1, v_b, o_b, first=False)
            return carry

        lax.fori_loop(1, n_full // 2, body, 0)
        drain_out(o_a)
        drain_out(o_b)

        col = n_full * 32 + wid

        @pl.when(wid < 4)
        def _tail_full():
            fire_in(col, v_a, 128)
            drain_in(v_a, 128)
            transpose(v_a, o_a, 128)
            fire_out(col, o_a, 128)
            drain_out(o_a)

        @pl.when(wid == 4)
        def _tail_partial():
            fire_in(col, v_a, 64)
            drain_in(v_a, 64)
            transpose(v_a, o_a, 64)
            fire_out(col, o_a, 64)
            pltpu.make_async_copy(o_a.at[pl.ds(0, 1024)],
                                  lin_hbm.at[pl.ds(0, 1024)], wsem).wait()

    return relin


def _make_gather():
    mesh = plsc.VectorSubcoreMesh(core_axis_name="c", subcore_axis_name="s")
    n_out = _S * D_EMBEDDING * _B

    @functools.partial(
        pl.kernel,
        mesh=mesh,
        out_type=jax.ShapeDtypeStruct((n_out,), jnp.float32),
        compiler_params=pltpu.CompilerParams(
            use_tc_tiling_on_sc=False, needs_layout_passes=False),
        scratch_types=[
            pltpu.VMEM((_S, _LANE), jnp.int32),
            pltpu.VMEM((_GS * _LANE, D_EMBEDDING), jnp.float32),
            pltpu.VMEM((_GS * _LANE, D_EMBEDDING), jnp.float32),
            pltpu.VMEM((_GS * 2048,), jnp.float32),
            pltpu.VMEM((_GS * 2048,), jnp.float32),
            pltpu.SemaphoreType.DMA,
            pltpu.SemaphoreType.DMA,
        ],
    )
    def gather(table_hbm, idxt_hbm, out_hbm, idxv, r_a, r_b, stg_a, stg_b,
               gsem, wsem):
        wid = lax.axis_index("s") * _NC + lax.axis_index("c")
        iota = lax.iota(jnp.int32, 16)

        pltpu.sync_copy(idxt_hbm.at[:, pl.ds(wid * _LANE, _LANE)], idxv)

        def fire_g(g, rbuf):
            for j in range(_GS):
                pltpu.async_copy(
                    table_hbm.at[idxv.at[g * _GS + j]],
                    rbuf.at[pl.ds(j * _LANE, _LANE)], gsem)

        def drain_g(rbuf):
            for j in range(_GS):
                pltpu.make_async_copy(
                    table_hbm.at[idxv.at[0]],
                    rbuf.at[pl.ds(j * _LANE, _LANE)], gsem).wait()

        rows = [iota + c * 16 for c in range(8)]

        def transpose(rbuf, stg):
            @plsc.parallel_loop(0, _GS * D_EMBEDDING, 1, unroll=2)
            def _(i):
                j = i >> 4
                d = i & 15
                col = jnp.zeros((16,), jnp.int32) + d
                base = j * 2048 + d * 128
                joff = j << 7
                for c in range(8):
                    v = plsc.load_gather(rbuf, [rows[c] + joff, col])
                    stg[pl.ds(base + c * 16, 16)] = v

        def fire_w(g, stg):
            for j in range(_GS):
                s = g * _GS + j
                for dt in range(2):
                    pltpu.async_copy(
                        stg.at[pl.ds(j * 2048 + dt * 1024, 1024)],
                        out_hbm.at[pl.ds(((2 * s + dt) * 32 + wid) * 1024, 1024)],
                        wsem)

        def drain_w(stg):
            for j in range(_GS):
                for dt in range(2):
                    pltpu.make_async_copy(
                        stg.at[pl.ds(j * 2048 + dt * 1024, 1024)],
                        out_hbm.at[pl.ds(0, 1024)], wsem).wait()

        def process(g, rbuf, stg, first):
            if not first:
                drain_w(stg)
            fire_g(g, rbuf)
            drain_g(rbuf)
            transpose(rbuf, stg)
            fire_w(g, stg)

        process(0, r_a, stg_a, first=True)
        process(1, r_b, stg_b, first=True)

        def body(i, carry):
            process(2 * i, r_a, stg_a, first=False)
            process(2 * i + 1, r_b, stg_b, first=False)
            return carry

        lax.fori_loop(1, _NG // 2, body, 0)
        drain_w(stg_a)
        drain_w(stg_b)

    return gather


def kernel(x, table):
    idx_t = _quantize_t(x)
    lin = _make_relinearize()(table.T).reshape(_NROWS_PAD, 16)
    flat = _make_gather()(lin, idx_t)
    return (flat.reshape(_S, 2, 32, 8, _LANE)
            .transpose(2, 4, 0, 1, 3)
            .reshape(_B, _S, D_EMBEDDING))


# speedup vs baseline: 2.6796x; 1.1842x over previous
import functools

import jax
import jax.numpy as jnp
from jax import lax
from jax.experimental import pallas as pl
from jax.experimental.pallas import tpu as pltpu
from jax.experimental.pallas import tpu_sc as plsc

N_EMBEDDINGS = 1000000
D_EMBEDDING = 16

_NC = 2
_NS = 16
_NW = _NC * _NS

_B = 4096
_S = 200
_LANE = 128
_GS = 4
_NG = _S // _GS


def _quantize_body(x_ref, idx_ref):
    x = x_ref[...]
    m = jnp.max(x, axis=1, keepdims=True)
    xn = x / m * float(N_EMBEDDINGS - 1)
    xn = jnp.where(xn < 0, 0.0, xn)
    idx_ref[...] = xn.astype(jnp.int32).T


def _quantize_t(x):
    return pl.pallas_call(
        _quantize_body,
        out_shape=jax.ShapeDtypeStruct((_S, _B), jnp.int32),
    )(x)


_NTILE = 7813
_NROWS_PAD = _NTILE * 128


def _make_relinearize():
    mesh = plsc.VectorSubcoreMesh(core_axis_name="c", subcore_axis_name="s")
    batch = 4
    n_step = 61

    @functools.partial(
        pl.kernel,
        mesh=mesh,
        out_type=jax.ShapeDtypeStruct((_NROWS_PAD * 16,), jnp.float32),
        compiler_params=pltpu.CompilerParams(
            use_tc_tiling_on_sc=True, needs_layout_passes=False),
        scratch_types=[
            pltpu.VMEM((16, batch * 128), jnp.float32),
            pltpu.VMEM((16, batch * 128), jnp.float32),
            pltpu.VMEM((batch * 2048,), jnp.float32),
            pltpu.VMEM((batch * 2048,), jnp.float32),
            pltpu.VMEM((16, 128), jnp.float32),
            pltpu.VMEM((2048,), jnp.float32),
            pltpu.SemaphoreType.DMA,
            pltpu.SemaphoreType.DMA,
        ],
    )
    def relin(tt_hbm, tail_hbm, lin_hbm, v_a, v_b, o_a, o_b, v_t, o_t, isem, wsem):
        wid = lax.axis_index("s") * _NC + lax.axis_index("c")
        iota = lax.iota(jnp.int32, 16)
        base = 244 * wid + jnp.minimum(wid, 4)

        def fire_in(lane, vbuf, width):
            for dt in range(2):
                pltpu.async_copy(
                    tt_hbm.at[pl.ds(dt * 8, 8), pl.ds(lane, width)],
                    vbuf.at[pl.ds(dt * 8, 8), pl.ds(0, width)], isem)

        def drain_in(vbuf, width):
            for dt in range(2):
                pltpu.make_async_copy(
                    tt_hbm.at[pl.ds(0, 8), pl.ds(0, width)],
                    vbuf.at[pl.ds(dt * 8, 8), pl.ds(0, width)], isem).wait()

        def transpose(vbuf, obuf, width):
            @plsc.parallel_loop(0, width, 1, unroll=8)
            def _(r0):
                v = plsc.load_gather(vbuf, [iota, jnp.zeros((16,), jnp.int32) + r0])
                obuf[pl.ds(r0 * 16, 16)] = v

        def fire_out(lane, obuf, width):
            pltpu.async_copy(obuf.at[pl.ds(0, width * 16)],
                             lin_hbm.at[pl.ds(lane * 16, width * 16)], wsem)

        def drain_out(obuf, width):
            pltpu.make_async_copy(obuf.at[pl.ds(0, width * 16)],
                                  lin_hbm.at[pl.ds(0, width * 16)], wsem).wait()

        def process(t, vbuf, obuf, first):
            lane = (base + t * batch) * 128
            if not first:
                drain_out(obuf, batch * 128)
            fire_in(lane, vbuf, batch * 128)
            drain_in(vbuf, batch * 128)
            transpose(vbuf, obuf, batch * 128)
            fire_out(lane, obuf, batch * 128)

        process(0, v_a, o_a, first=True)
        process(1, v_b, o_b, first=True)

        def body(i, carry):
            process(2 * i, v_a, o_a, first=False)
            process(2 * i + 1, v_b, o_b, first=False)
            return carry

        lax.fori_loop(1, n_step // 2, body, 0)
        process(n_step - 1, v_a, o_a, first=False)
        drain_out(o_b, batch * 128)
        drain_out(o_a, batch * 128)

        @pl.when(wid < 4)
        def _tail_full():
            lane = (base + n_step * batch) * 128
            for dt in range(2):
                pltpu.async_copy(
                    tt_hbm.at[pl.ds(dt * 8, 8), pl.ds(lane, 128)],
                    v_t.at[pl.ds(dt * 8, 8)], isem)
            for dt in range(2):
                pltpu.make_async_copy(
                    tt_hbm.at[pl.ds(0, 8), pl.ds(0, 128)],
                    v_t.at[pl.ds(dt * 8, 8)], isem).wait()

            @plsc.parallel_loop(0, 128, 1, unroll=8)
            def _(r0):
                v = plsc.load_gather(v_t, [iota, jnp.zeros((16,), jnp.int32) + r0])
                o_t[pl.ds(r0 * 16, 16)] = v
            pltpu.async_copy(o_t, lin_hbm.at[pl.ds(lane * 16, 2048)], wsem)
            pltpu.make_async_copy(o_t, lin_hbm.at[pl.ds(0, 2048)], wsem).wait()

        @pl.when(wid == 31)
        def _tail_partial():
            pltpu.sync_copy(tail_hbm, o_t.at[pl.ds(0, 1024)])
            pltpu.sync_copy(o_t.at[pl.ds(0, 1024)],
                            lin_hbm.at[pl.ds((_NTILE - 1) * 2048, 1024)])

    return relin


def _make_gather():
    mesh = plsc.VectorSubcoreMesh(core_axis_name="c", subcore_axis_name="s")
    n_out = _S * D_EMBEDDING * _B

    @functools.partial(
        pl.kernel,
        mesh=mesh,
        out_type=jax.ShapeDtypeStruct((n_out,), jnp.float32),
        compiler_params=pltpu.CompilerParams(
            use_tc_tiling_on_sc=False, needs_layout_passes=False),
        scratch_types=[
            pltpu.VMEM((_S, _LANE), jnp.int32),
            pltpu.VMEM((_GS * _LANE, D_EMBEDDING), jnp.float32),
            pltpu.VMEM((_GS * _LANE, D_EMBEDDING), jnp.float32),
            pltpu.VMEM((_GS * 2048,), jnp.float32),
            pltpu.VMEM((_GS * 2048,), jnp.float32),
            pltpu.SemaphoreType.DMA,
            pltpu.SemaphoreType.DMA,
        ],
    )
    def gather(table_hbm, idxt_hbm, out_hbm, idxv, r_a, r_b, stg_a, stg_b,
               gsem, wsem):
        wid = lax.axis_index("s") * _NC + lax.axis_index("c")
        iota = lax.iota(jnp.int32, 16)

        pltpu.sync_copy(idxt_hbm.at[:, pl.ds(wid * _LANE, _LANE)], idxv)

        def fire_g(g, rbuf):
            for j in range(_GS):
                pltpu.async_copy(
                    table_hbm.at[idxv.at[g * _GS + j]],
                    rbuf.at[pl.ds(j * _LANE, _LANE)], gsem)

        def drain_g(rbuf):
            for j in range(_GS):
                pltpu.make_async_copy(
                    table_hbm.at[idxv.at[0]],
                    rbuf.at[pl.ds(j * _LANE, _LANE)], gsem).wait()

        rows = [iota + c * 16 for c in range(8)]

        def transpose(rbuf, stg):
            @plsc.parallel_loop(0, _GS * D_EMBEDDING, 1, unroll=2)
            def _(i):
                j = i >> 4
                d = i & 15
                col = jnp.zeros((16,), jnp.int32) + d
                base = j * 2048 + d * 128
                joff = j << 7
                for c in range(8):
                    v = plsc.load_gather(rbuf, [rows[c] + joff, col])
                    stg[pl.ds(base + c * 16, 16)] = v

        def fire_w(g, stg):
            for j in range(_GS):
                s = g * _GS + j
                for dt in range(2):
                    pltpu.async_copy(
                        stg.at[pl.ds(j * 2048 + dt * 1024, 1024)],
                        out_hbm.at[pl.ds(((2 * s + dt) * 32 + wid) * 1024, 1024)],
                        wsem)

        def drain_w(stg):
            for j in range(_GS):
                for dt in range(2):
                    pltpu.make_async_copy(
                        stg.at[pl.ds(j * 2048 + dt * 1024, 1024)],
                        out_hbm.at[pl.ds(0, 1024)], wsem).wait()

        def process(g, rbuf, stg, first):
            if not first:
                drain_w(stg)
            fire_g(g, rbuf)
            drain_g(rbuf)
            transpose(rbuf, stg)
            fire_w(g, stg)

        process(0, r_a, stg_a, first=True)
        process(1, r_b, stg_b, first=True)

        def body(i, carry):
            process(2 * i, r_a, stg_a, first=False)
            process(2 * i + 1, r_b, stg_b, first=False)
            return carry

        lax.fori_loop(1, _NG // 2, body, 0)
        drain_w(stg_a)
        drain_w(stg_b)

    return gather


def kernel(x, table):
    idx_t = _quantize_t(x)
    tail = table[(_NTILE - 1) * 128:, :].reshape(1024)
    lin = _make_relinearize()(table.T, tail).reshape(_NROWS_PAD, 16)
    flat = _make_gather()(lin, idx_t)
    return (flat.reshape(_S, 2, 32, 8, _LANE)
            .transpose(2, 4, 0, 1, 3)
            .reshape(_B, _S, D_EMBEDDING))


# speedup vs baseline: 2.6811x; 1.0006x over previous
import functools

import jax
import jax.numpy as jnp
from jax import lax
from jax.experimental import pallas as pl
from jax.experimental.pallas import tpu as pltpu
from jax.experimental.pallas import tpu_sc as plsc

N_EMBEDDINGS = 1000000
D_EMBEDDING = 16

_NC = 2
_NS = 16
_NW = _NC * _NS

_B = 4096
_S = 200
_LANE = 128
_GS = 4
_NG = _S // _GS


def _quantize_body(x_ref, idx_ref):
    x = x_ref[...]
    m = jnp.max(x, axis=1, keepdims=True)
    xn = x / m * float(N_EMBEDDINGS - 1)
    xn = jnp.where(xn < 0, 0.0, xn)
    idx_ref[...] = xn.astype(jnp.int32).T


def _quantize_t(x):
    return pl.pallas_call(
        _quantize_body,
        out_shape=jax.ShapeDtypeStruct((_S, _B), jnp.int32),
    )(x)


_NTILE = 7813
_NROWS_PAD = _NTILE * 128


def _make_relinearize():
    mesh = plsc.VectorSubcoreMesh(core_axis_name="c", subcore_axis_name="s")
    batch = 4
    n_step = 61

    @functools.partial(
        pl.kernel,
        mesh=mesh,
        out_type=jax.ShapeDtypeStruct((_NROWS_PAD * 16,), jnp.float32),
        compiler_params=pltpu.CompilerParams(
            use_tc_tiling_on_sc=True, needs_layout_passes=False),
        scratch_types=[
            pltpu.VMEM((16, batch * 128 + 1), jnp.float32),
            pltpu.VMEM((16, batch * 128 + 1), jnp.float32),
            pltpu.VMEM((batch * 2048,), jnp.float32),
            pltpu.VMEM((batch * 2048,), jnp.float32),
            pltpu.VMEM((16, 128), jnp.float32),
            pltpu.VMEM((2048,), jnp.float32),
            pltpu.SemaphoreType.DMA,
            pltpu.SemaphoreType.DMA,
        ],
    )
    def relin(tt_hbm, tail_hbm, lin_hbm, v_a, v_b, o_a, o_b, v_t, o_t, isem, wsem):
        wid = lax.axis_index("s") * _NC + lax.axis_index("c")
        iota = lax.iota(jnp.int32, 16)
        base = 244 * wid + jnp.minimum(wid, 4)

        def fire_in(lane, vbuf, width):
            for dt in range(2):
                pltpu.async_copy(
                    tt_hbm.at[pl.ds(dt * 8, 8), pl.ds(lane, width)],
                    vbuf.at[pl.ds(dt * 8, 8), pl.ds(0, width)], isem)

        def drain_in(vbuf, width):
            for dt in range(2):
                pltpu.make_async_copy(
                    tt_hbm.at[pl.ds(0, 8), pl.ds(0, width)],
                    vbuf.at[pl.ds(dt * 8, 8), pl.ds(0, width)], isem).wait()

        def transpose(vbuf, obuf, width):
            @plsc.parallel_loop(0, width, 1, unroll=8)
            def _(r0):
                v = plsc.load_gather(vbuf, [iota, jnp.zeros((16,), jnp.int32) + r0])
                obuf[pl.ds(r0 * 16, 16)] = v

        def fire_out(lane, obuf, width):
            pltpu.async_copy(obuf.at[pl.ds(0, width * 16)],
                             lin_hbm.at[pl.ds(lane * 16, width * 16)], wsem)

        def drain_out(obuf, width):
            pltpu.make_async_copy(obuf.at[pl.ds(0, width * 16)],
                                  lin_hbm.at[pl.ds(0, width * 16)], wsem).wait()

        def process(t, vbuf, obuf, first):
            lane = (base + t * batch) * 128
            if not first:
                drain_out(obuf, batch * 128)
            fire_in(lane, vbuf, batch * 128)
            drain_in(vbuf, batch * 128)
            transpose(vbuf, obuf, batch * 128)
            fire_out(lane, obuf, batch * 128)

        process(0, v_a, o_a, first=True)
        process(1, v_b, o_b, first=True)

        def body(i, carry):
            process(2 * i, v_a, o_a, first=False)
            process(2 * i + 1, v_b, o_b, first=False)
            return carry

        lax.fori_loop(1, n_step // 2, body, 0)
        process(n_step - 1, v_a, o_a, first=False)
        drain_out(o_b, batch * 128)
        drain_out(o_a, batch * 128)

        @pl.when(wid < 4)
        def _tail_full():
            lane = (base + n_step * batch) * 128
            for dt in range(2):
                pltpu.async_copy(
                    tt_hbm.at[pl.ds(dt * 8, 8), pl.ds(lane, 128)],
                    v_t.at[pl.ds(dt * 8, 8)], isem)
            for dt in range(2):
                pltpu.make_async_copy(
                    tt_hbm.at[pl.ds(0, 8), pl.ds(0, 128)],
                    v_t.at[pl.ds(dt * 8, 8)], isem).wait()

            @plsc.parallel_loop(0, 128, 1, unroll=8)
            def _(r0):
                v = plsc.load_gather(v_t, [iota, jnp.zeros((16,), jnp.int32) + r0])
                o_t[pl.ds(r0 * 16, 16)] = v
            pltpu.async_copy(o_t, lin_hbm.at[pl.ds(lane * 16, 2048)], wsem)
            pltpu.make_async_copy(o_t, lin_hbm.at[pl.ds(0, 2048)], wsem).wait()

        @pl.when(wid == 31)
        def _tail_partial():
            pltpu.sync_copy(tail_hbm, o_t.at[pl.ds(0, 1024)])
            pltpu.sync_copy(o_t.at[pl.ds(0, 1024)],
                            lin_hbm.at[pl.ds((_NTILE - 1) * 2048, 1024)])

    return relin


def _make_gather():
    mesh = plsc.VectorSubcoreMesh(core_axis_name="c", subcore_axis_name="s")
    n_out = _S * D_EMBEDDING * _B

    @functools.partial(
        pl.kernel,
        mesh=mesh,
        out_type=jax.ShapeDtypeStruct((n_out,), jnp.float32),
        compiler_params=pltpu.CompilerParams(
            use_tc_tiling_on_sc=False, needs_layout_passes=False),
        scratch_types=[
            pltpu.VMEM((_S, _LANE), jnp.int32),
            pltpu.VMEM((_GS * _LANE, D_EMBEDDING), jnp.float32),
            pltpu.VMEM((_GS * _LANE, D_EMBEDDING), jnp.float32),
            pltpu.VMEM((_GS * 2048,), jnp.float32),
            pltpu.VMEM((_GS * 2048,), jnp.float32),
            pltpu.SemaphoreType.DMA,
            pltpu.SemaphoreType.DMA,
        ],
    )
    def gather(table_hbm, idxt_hbm, out_hbm, idxv, r_a, r_b, stg_a, stg_b,
               gsem, wsem):
        wid = lax.axis_index("s") * _NC + lax.axis_index("c")
        iota = lax.iota(jnp.int32, 16)

        pltpu.sync_copy(idxt_hbm.at[:, pl.ds(wid * _LANE, _LANE)], idxv)

        def fire_g(g, rbuf):
            for j in range(_GS):
                pltpu.async_copy(
                    table_hbm.at[idxv.at[g * _GS + j]],
                    rbuf.at[pl.ds(j * _LANE, _LANE)], gsem)

        def drain_g(rbuf):
            for j in range(_GS):
                pltpu.make_async_copy(
                    table_hbm.at[idxv.at[0]],
                    rbuf.at[pl.ds(j * _LANE, _LANE)], gsem).wait()

        rows = [iota + c * 16 for c in range(8)]

        def transpose(rbuf, stg):
            @plsc.parallel_loop(0, _GS * D_EMBEDDING, 1, unroll=2)
            def _(i):
                j = i >> 4
                d = i & 15
                col = jnp.zeros((16,), jnp.int32) + d
                base = j * 2048 + d * 128
                joff = j << 7
                for c in range(8):
                    v = plsc.load_gather(rbuf, [rows[c] + joff, col])
                    stg[pl.ds(base + c * 16, 16)] = v

        def fire_w(g, stg):
            for j in range(_GS):
                s = g * _GS + j
                for dt in range(2):
                    pltpu.async_copy(
                        stg.at[pl.ds(j * 2048 + dt * 1024, 1024)],
                        out_hbm.at[pl.ds(((2 * s + dt) * 32 + wid) * 1024, 1024)],
                        wsem)

        def drain_w(stg):
            for j in range(_GS):
                for dt in range(2):
                    pltpu.make_async_copy(
                        stg.at[pl.ds(j * 2048 + dt * 1024, 1024)],
                        out_hbm.at[pl.ds(0, 1024)], wsem).wait()

        def process(g, rbuf, stg, first):
            if not first:
                drain_w(stg)
            fire_g(g, rbuf)
            drain_g(rbuf)
            transpose(rbuf, stg)
            fire_w(g, stg)

        process(0, r_a, stg_a, first=True)
        process(1, r_b, stg_b, first=True)

        def body(i, carry):
            process(2 * i, r_a, stg_a, first=False)
            process(2 * i + 1, r_b, stg_b, first=False)
            return carry

        lax.fori_loop(1, _NG // 2, body, 0)
        drain_w(stg_a)
        drain_w(stg_b)

    return gather


def kernel(x, table):
    idx_t = _quantize_t(x)
    tail = table[(_NTILE - 1) * 128:, :].reshape(1024)
    lin = _make_relinearize()(table.T, tail).reshape(_NROWS_PAD, 16)
    flat = _make_gather()(lin, idx_t)
    return (flat.reshape(_S, 2, 32, 8, _LANE)
            .transpose(2, 4, 0, 1, 3)
            .reshape(_B, _S, D_EMBEDDING))


# speedup vs baseline: 3.5696x; 1.3314x over previous
import functools

import jax
import jax.numpy as jnp
from jax import lax
from jax.experimental import pallas as pl
from jax.experimental.pallas import tpu as pltpu
from jax.experimental.pallas import tpu_sc as plsc

N_EMBEDDINGS = 1000000
D_EMBEDDING = 16

_NC = 2
_NS = 16
_NW = _NC * _NS

_B = 4096
_S = 200
_LANE = 128
_GS = 4
_NG = _S // _GS


def _quantize_body(x_ref, idx_ref):
    x = x_ref[...]
    m = jnp.max(x, axis=1, keepdims=True)
    xn = x / m * float(N_EMBEDDINGS - 1)
    xn = jnp.where(xn < 0, 0.0, xn)
    idx_ref[...] = xn.astype(jnp.int32).T


def _quantize_t(x):
    return pl.pallas_call(
        _quantize_body,
        out_shape=jax.ShapeDtypeStruct((_S, _B), jnp.int32),
    )(x)


_NTILE = 7813
_NROWS_PAD = _NTILE * 128


def _make_relinearize():
    mesh = plsc.VectorSubcoreMesh(core_axis_name="c", subcore_axis_name="s")
    batch = 4
    n_step = 61

    @functools.partial(
        pl.kernel,
        mesh=mesh,
        out_type=jax.ShapeDtypeStruct((_NROWS_PAD * 16,), jnp.float32),
        compiler_params=pltpu.CompilerParams(
            use_tc_tiling_on_sc=True, needs_layout_passes=False),
        scratch_types=[
            pltpu.VMEM((16, batch * 128 + 1), jnp.float32),
            pltpu.VMEM((16, batch * 128 + 1), jnp.float32),
            pltpu.VMEM((batch * 2048,), jnp.float32),
            pltpu.VMEM((batch * 2048,), jnp.float32),
            pltpu.VMEM((16, 128), jnp.float32),
            pltpu.VMEM((2048,), jnp.float32),
            pltpu.SemaphoreType.DMA,
            pltpu.SemaphoreType.DMA,
        ],
    )
    def relin(tt_hbm, tail_hbm, lin_hbm, v_a, v_b, o_a, o_b, v_t, o_t, isem, wsem):
        wid = lax.axis_index("s") * _NC + lax.axis_index("c")
        iota = lax.iota(jnp.int32, 16)
        base = 244 * wid + jnp.minimum(wid, 4)

        def fire_in(lane, vbuf, width):
            for dt in range(2):
                pltpu.async_copy(
                    tt_hbm.at[pl.ds(dt * 8, 8), pl.ds(lane, width)],
                    vbuf.at[pl.ds(dt * 8, 8), pl.ds(0, width)], isem)

        def drain_in(vbuf, width):
            for dt in range(2):
                pltpu.make_async_copy(
                    tt_hbm.at[pl.ds(0, 8), pl.ds(0, width)],
                    vbuf.at[pl.ds(dt * 8, 8), pl.ds(0, width)], isem).wait()

        def transpose(vbuf, obuf, width):
            @plsc.parallel_loop(0, width, 8, unroll=2)
            def _(r0):
                for k in range(8):
                    v = plsc.load_gather(
                        vbuf, [iota, jnp.zeros((16,), jnp.int32) + (r0 + k)])
                    obuf[pl.ds((r0 + k) * 16, 16)] = v

        def fire_out(lane, obuf, width):
            pltpu.async_copy(obuf.at[pl.ds(0, width * 16)],
                             lin_hbm.at[pl.ds(lane * 16, width * 16)], wsem)

        def drain_out(obuf, width):
            pltpu.make_async_copy(obuf.at[pl.ds(0, width * 16)],
                                  lin_hbm.at[pl.ds(0, width * 16)], wsem).wait()

        def lane_of(t):
            return (base + t * batch) * 128

        def process(t, vbuf, obuf, fire_ahead, first):
            drain_in(vbuf, batch * 128)
            if not first:
                drain_out(obuf, batch * 128)
            transpose(vbuf, obuf, batch * 128)
            fire_out(lane_of(t), obuf, batch * 128)
            if fire_ahead:
                fire_in(lane_of(t + 2), vbuf, batch * 128)

        fire_in(lane_of(0), v_a, batch * 128)
        fire_in(lane_of(1), v_b, batch * 128)
        process(0, v_a, o_a, fire_ahead=True, first=True)
        process(1, v_b, o_b, fire_ahead=True, first=True)

        def body(i, carry):
            process(2 * i, v_a, o_a, fire_ahead=True, first=False)
            process(2 * i + 1, v_b, o_b, fire_ahead=True, first=False)
            return carry

        lax.fori_loop(1, 29, body, 0)
        process(58, v_a, o_a, fire_ahead=False, first=False)
        fire_in(lane_of(60), v_a, batch * 128)
        process(59, v_b, o_b, fire_ahead=False, first=False)
        process(60, v_a, o_a, fire_ahead=False, first=False)
        drain_out(o_b, batch * 128)
        drain_out(o_a, batch * 128)

        @pl.when(wid < 4)
        def _tail_full():
            lane = (base + n_step * batch) * 128
            for dt in range(2):
                pltpu.async_copy(
                    tt_hbm.at[pl.ds(dt * 8, 8), pl.ds(lane, 128)],
                    v_t.at[pl.ds(dt * 8, 8)], isem)
            for dt in range(2):
                pltpu.make_async_copy(
                    tt_hbm.at[pl.ds(0, 8), pl.ds(0, 128)],
                    v_t.at[pl.ds(dt * 8, 8)], isem).wait()

            @plsc.parallel_loop(0, 128, 1, unroll=8)
            def _(r0):
                v = plsc.load_gather(v_t, [iota, jnp.zeros((16,), jnp.int32) + r0])
                o_t[pl.ds(r0 * 16, 16)] = v
            pltpu.async_copy(o_t, lin_hbm.at[pl.ds(lane * 16, 2048)], wsem)
            pltpu.make_async_copy(o_t, lin_hbm.at[pl.ds(0, 2048)], wsem).wait()

        @pl.when(wid == 31)
        def _tail_partial():
            pltpu.sync_copy(tail_hbm, o_t.at[pl.ds(0, 1024)])
            pltpu.sync_copy(o_t.at[pl.ds(0, 1024)],
                            lin_hbm.at[pl.ds((_NTILE - 1) * 2048, 1024)])

    return relin


def _make_gather():
    mesh = plsc.VectorSubcoreMesh(core_axis_name="c", subcore_axis_name="s")
    n_out = _S * D_EMBEDDING * _B

    @functools.partial(
        pl.kernel,
        mesh=mesh,
        out_type=jax.ShapeDtypeStruct((n_out,), jnp.float32),
        compiler_params=pltpu.CompilerParams(
            use_tc_tiling_on_sc=False, needs_layout_passes=False),
        scratch_types=[
            pltpu.VMEM((_S, _LANE), jnp.int32),
            pltpu.VMEM((_GS * _LANE, D_EMBEDDING), jnp.float32),
            pltpu.VMEM((_GS * _LANE, D_EMBEDDING), jnp.float32),
            pltpu.VMEM((_GS * 2048,), jnp.float32),
            pltpu.VMEM((_GS * 2048,), jnp.float32),
            pltpu.SemaphoreType.DMA,
            pltpu.SemaphoreType.DMA,
        ],
    )
    def gather(table_hbm, idxt_hbm, out_hbm, idxv, r_a, r_b, stg_a, stg_b,
               gsem, wsem):
        wid = lax.axis_index("s") * _NC + lax.axis_index("c")
        iota = lax.iota(jnp.int32, 16)

        pltpu.sync_copy(idxt_hbm.at[:, pl.ds(wid * _LANE, _LANE)], idxv)

        def fire_g(g, rbuf):
            for j in range(_GS):
                pltpu.async_copy(
                    table_hbm.at[idxv.at[g * _GS + j]],
                    rbuf.at[pl.ds(j * _LANE, _LANE)], gsem)

        def drain_g(rbuf):
            for j in range(_GS):
                pltpu.make_async_copy(
                    table_hbm.at[idxv.at[0]],
                    rbuf.at[pl.ds(j * _LANE, _LANE)], gsem).wait()

        rows = [iota + c * 16 for c in range(8)]

        def transpose(rbuf, stg):
            @plsc.parallel_loop(0, _GS * D_EMBEDDING, 1, unroll=2)
            def _(i):
                j = i >> 4
                d = i & 15
                col = jnp.zeros((16,), jnp.int32) + d
                base = j * 2048 + d * 128
                joff = j << 7
                for c in range(8):
                    v = plsc.load_gather(rbuf, [rows[c] + joff, col])
                    stg[pl.ds(base + c * 16, 16)] = v

        def fire_w(g, stg):
            for j in range(_GS):
                s = g * _GS + j
                for dt in range(2):
                    pltpu.async_copy(
                        stg.at[pl.ds(j * 2048 + dt * 1024, 1024)],
                        out_hbm.at[pl.ds(((2 * s + dt) * 32 + wid) * 1024, 1024)],
                        wsem)

        def drain_w(stg):
            for j in range(_GS):
                for dt in range(2):
                    pltpu.make_async_copy(
                        stg.at[pl.ds(j * 2048 + dt * 1024, 1024)],
                        out_hbm.at[pl.ds(0, 1024)], wsem).wait()

        def process(g, rbuf, stg, fire_ahead, first):
            drain_g(rbuf)
            if not first:
                drain_w(stg)
            transpose(rbuf, stg)
            fire_w(g, stg)
            if fire_ahead:
                fire_g(g + 2, rbuf)

        fire_g(0, r_a)
        fire_g(1, r_b)
        process(0, r_a, stg_a, fire_ahead=True, first=True)
        process(1, r_b, stg_b, fire_ahead=True, first=True)

        def body(i, carry):
            process(2 * i, r_a, stg_a, fire_ahead=True, first=False)
            process(2 * i + 1, r_b, stg_b, fire_ahead=True, first=False)
            return carry

        lax.fori_loop(1, _NG // 2 - 1, body, 0)
        process(_NG - 2, r_a, stg_a, fire_ahead=False, first=False)
        process(_NG - 1, r_b, stg_b, fire_ahead=False, first=False)
        drain_w(stg_a)
        drain_w(stg_b)

    return gather


def kernel(x, table):
    idx_t = _quantize_t(x)
    tail = table[(_NTILE - 1) * 128:, :].reshape(1024)
    lin = _make_relinearize()(table.T, tail).reshape(_NROWS_PAD, 16)
    flat = _make_gather()(lin, idx_t)
    return (flat.reshape(_S, 2, 32, 8, _LANE)
            .transpose(2, 4, 0, 1, 3)
            .reshape(_B, _S, D_EMBEDDING))
